# Initial kernel scaffold; baseline (speedup 1.0000x reference)
import jax
import jax.numpy as jnp
from jax.experimental import pallas as pl


def kernel(U, V, W_u, W_v):
    raise NotImplementedError("write your pallas kernel here")


# speedup vs baseline: 1.2218x; 1.2218x over previous
import functools

import jax
import jax.numpy as jnp
from jax import lax
from jax.experimental import pallas as pl
from jax.experimental.pallas import tpu as pltpu
from jax.experimental.pallas import tpu_sc as plsc

NC = 2
NS = 16
NW = NC * NS
L = 16


def _make_sc_kernel(B, NNZ, Du, Dv):
    RPW = B // NW
    CHUNKS = RPW // L
    mesh = plsc.VectorSubcoreMesh(core_axis_name="c", subcore_axis_name="s")

    @functools.partial(
        pl.kernel,
        mesh=mesh,
        compiler_params=pltpu.CompilerParams(needs_layout_passes=False),
        out_type=[jax.ShapeDtypeStruct((B,), jnp.float32)] * 4,
        scratch_types=[
            pltpu.VMEM((RPW * NNZ,), jnp.int32),
            pltpu.VMEM((RPW * NNZ,), jnp.int32),
            pltpu.VMEM((RPW * NNZ,), jnp.float32),
            pltpu.VMEM((RPW * NNZ,), jnp.float32),
            pltpu.VMEM((RPW,), jnp.float32),
            pltpu.SemaphoreType.DMA,
            pltpu.SemaphoreType.DMA,
            pltpu.SemaphoreType.DMA,
        ],
    )
    def body(u_hbm, v_hbm, wu_hbm, wv_hbm,
             p_hbm, un_hbm, q_hbm, vn_hbm,
             uidx_v, vidx_v, uval_v, vval_v, acc_v,
             sem_idx, sem_gu, sem_gv):
        wid = lax.axis_index("s") * NC + lax.axis_index("c")
        ibase = wid * (RPW * NNZ)
        obase = wid * RPW

        cp_u = pltpu.async_copy(u_hbm.at[pl.ds(ibase, RPW * NNZ)], uidx_v, sem_idx)
        cp_v = pltpu.async_copy(v_hbm.at[pl.ds(ibase, RPW * NNZ)], vidx_v, sem_idx)
        cp_u.wait()
        g_u = pltpu.async_copy(wu_hbm.at[uidx_v], uval_v, sem_gu)
        cp_v.wait()
        g_v = pltpu.async_copy(wv_hbm.at[vidx_v], vval_v, sem_gv)

        lane = lax.iota(jnp.int32, L) * NNZ

        def count_chunk(idx_ref, c, _):
            flat = c * (L * NNZ) + lane
            acc = jnp.zeros((L,), jnp.float32)
            for j in range(NNZ):
                iv = plsc.load_gather(idx_ref, [flat + j])
                acc = acc + jnp.where(iv != 0, 1.0, 0.0).astype(jnp.float32)
            acc_v[pl.ds(c * L, L)] = acc
            return _

        def sum_chunk(val_ref, c, _):
            flat = c * (L * NNZ) + lane
            acc = jnp.zeros((L,), jnp.float32)
            for j in range(NNZ):
                acc = acc + plsc.load_gather(val_ref, [flat + j])
            acc_v[pl.ds(c * L, L)] = acc
            return _

        lax.fori_loop(0, CHUNKS, functools.partial(count_chunk, uidx_v), 0)
        pltpu.sync_copy(acc_v, un_hbm.at[pl.ds(obase, RPW)])
        lax.fori_loop(0, CHUNKS, functools.partial(count_chunk, vidx_v), 0)
        pltpu.sync_copy(acc_v, vn_hbm.at[pl.ds(obase, RPW)])

        g_u.wait()
        lax.fori_loop(0, CHUNKS, functools.partial(sum_chunk, uval_v), 0)
        pltpu.sync_copy(acc_v, p_hbm.at[pl.ds(obase, RPW)])
        g_v.wait()
        lax.fori_loop(0, CHUNKS, functools.partial(sum_chunk, vval_v), 0)
        pltpu.sync_copy(acc_v, q_hbm.at[pl.ds(obase, RPW)])

    return body


def kernel(U, V, W_u, W_v):
    B, NNZ = U.shape
    Du = W_u.shape[0]
    Dv = W_v.shape[0]
    u_flat = U.astype(jnp.int32).reshape(-1)
    v_flat = V.astype(jnp.int32).reshape(-1)
    wu = W_u.reshape(-1)
    wv = W_v.reshape(-1)
    p, un, q, vn = _make_sc_kernel(B, NNZ, Du, Dv)(u_flat, v_flat, wu, wv)
    return p.reshape(B, 1), un, q.reshape(B, 1), vn
